# Initial kernel scaffold; baseline (speedup 1.0000x reference)
import jax
import jax.numpy as jnp
from jax.experimental import pallas as pl


def kernel(x, edge_index, adj_vals, W1, b1, W3, b3):
    raise NotImplementedError("write your pallas kernel here")


# speedup vs baseline: 15.4960x; 15.4960x over previous
import functools

import jax
import jax.numpy as jnp
from jax import lax
from jax.experimental import pallas as pl
from jax.experimental.pallas import tpu as pltpu
from jax.experimental.pallas import tpu_sc as plsc

LANES = 16
NT = 16
NC = 2
CH = 1024
NSUB = CH // 128


def _matmul1(x, w1, np_nodes):
    m, k = x.shape
    h = w1.shape[1]
    blk = 2048

    def body(x_ref, w_ref, o_ref):
        s = jnp.dot(x_ref[...], w_ref[...], preferred_element_type=jnp.float32)
        o_ref[0] = s[:, :LANES]
        o_ref[1] = s[:, LANES:]

    return pl.pallas_call(
        body,
        grid=(np_nodes // blk,),
        in_specs=[
            pl.BlockSpec((blk, k), lambda i: (i, 0)),
            pl.BlockSpec((k, h), lambda i: (0, 0)),
        ],
        out_specs=pl.BlockSpec((2, blk, LANES), lambda i: (0, i, 0)),
        out_shape=jax.ShapeDtypeStruct((2, np_nodes, LANES), jnp.float32),
    )(x, w1)


def _matmul2(agg, w3, b3, m):
    c = w3.shape[1]
    blk = 2000

    def body(a_ref, w_ref, b_ref, o_ref):
        s = jnp.concatenate([a_ref[0], a_ref[1]], axis=1)
        o_ref[...] = jnp.dot(s, w_ref[...], preferred_element_type=jnp.float32) + b_ref[...]

    return pl.pallas_call(
        body,
        grid=(m // blk,),
        in_specs=[
            pl.BlockSpec((2, blk, LANES), lambda i: (0, i, 0)),
            pl.BlockSpec((2 * LANES, c), lambda i: (0, 0)),
            pl.BlockSpec((1, c), lambda i: (0, 0)),
        ],
        out_specs=pl.BlockSpec((blk, c), lambda i: (i, 0)),
        out_shape=jax.ShapeDtypeStruct((m, c), jnp.float32),
    )(agg, w3, b3.reshape(1, c))


def _make_spmm(np_nodes, nch, fuse_bias_relu):
    rows_per_tile = np_nodes // NT
    rows_chunk = rows_per_tile // 16
    nrc = 16
    mesh = plsc.VectorSubcoreMesh(core_axis_name="c", subcore_axis_name="s")

    def body(sup_hbm, srcs_hbm, dsts_hbm, adjs_hbm, bias_hbm, out_hbm,
             src_v, dst_v, adj_v, rows_v, post_v, bias_v, acc_sh, sem):
        cid = lax.axis_index("c")
        tid = lax.axis_index("s")

        def zbody(i, carry):
            post_v[i] = jnp.zeros((LANES,), jnp.float32)
            return carry
        lax.fori_loop(0, rows_chunk, zbody, None)
        for r in range(nrc):
            pltpu.sync_copy(
                post_v,
                acc_sh.at[pl.ds(tid * rows_per_tile + r * rows_chunk, rows_chunk)])
        plsc.subcore_barrier()

        def cbody(ci, carry):
            pltpu.sync_copy(srcs_hbm.at[cid, tid, ci], src_v)
            pltpu.sync_copy(dsts_hbm.at[tid, ci], dst_v)
            pltpu.sync_copy(adjs_hbm.at[tid, ci], adj_v)
            copies = [
                pltpu.async_copy(
                    sup_hbm.at[src_v.at[pl.ds(j * 128, 128)]],
                    rows_v.at[pl.ds(j * 128, 128)],
                    sem,
                )
                for j in range(NSUB)
            ]
            for cp in copies:
                cp.wait()

            def sbody(g, inner):
                base = g * LANES
                adjv = adj_v[pl.ds(base, LANES)]
                for e in range(LANES):
                    rows_v[base + e] = rows_v[base + e] * adjv[e]
                return inner
            lax.fori_loop(0, CH // LANES, sbody, None, unroll=2)

            for j in range(NSUB):
                pltpu.sync_copy(rows_v.at[pl.ds(j * 128, 128)],
                                acc_sh.at[dst_v.at[j]], add=True)
            return carry
        lax.fori_loop(0, nch, cbody, None)
        plsc.subcore_barrier()

        if fuse_bias_relu:
            pltpu.sync_copy(bias_hbm, bias_v)
        for r in range(nrc):
            base = tid * rows_per_tile + r * rows_chunk
            if fuse_bias_relu:
                pltpu.sync_copy(acc_sh.at[pl.ds(base, rows_chunk)], post_v)
                bh = bias_v[cid]

                def pbody(e, carry):
                    post_v[e] = jnp.maximum(post_v[e] + bh, 0.0)
                    return carry
                lax.fori_loop(0, rows_chunk, pbody, None, unroll=8)
                pltpu.sync_copy(post_v, out_hbm.at[cid, pl.ds(base, rows_chunk)])
            else:
                pltpu.sync_copy(acc_sh.at[pl.ds(base, rows_chunk)],
                                out_hbm.at[cid, pl.ds(base, rows_chunk)])

    return pl.kernel(
        body,
        out_type=jax.ShapeDtypeStruct((NC, np_nodes, LANES), jnp.float32),
        mesh=mesh,
        compiler_params=pltpu.CompilerParams(use_tc_tiling_on_sc=False),
        scratch_types=[
            pltpu.VMEM((CH,), jnp.int32),
            pltpu.VMEM((NSUB, 128), jnp.int32),
            pltpu.VMEM((CH,), jnp.float32),
            pltpu.VMEM((CH, LANES), jnp.float32),
            pltpu.VMEM((rows_chunk, LANES), jnp.float32),
            pltpu.VMEM((NC, LANES), jnp.float32),
            pltpu.VMEM_SHARED((np_nodes, LANES), jnp.float32),
            pltpu.SemaphoreType.DMA,
        ],
    )


def kernel(x, edge_index, adj_vals, W1, b1, W3, b3):
    n = x.shape[0]
    e = edge_index.shape[1]
    np_nodes = -(-n // (NT * 32)) * (NT * 32)

    nch = -(-e // (NT * CH))
    ep = NT * CH * nch
    pad = ep - e
    dst = edge_index[0]
    src = edge_index[1]
    if pad:
        src = jnp.pad(src, (0, pad))
        dst = jnp.pad(dst, (0, pad))
        av = jnp.pad(adj_vals, (0, pad))
    else:
        av = adj_vals
    srcs = jnp.stack([src, src + np_nodes]).reshape(NC, NT, nch, CH)
    dsts = dst.reshape(NT, nch, NSUB, 128)
    adjs = av.reshape(NT, nch, CH)
    b1p = b1.reshape(NC, LANES)

    sup = _matmul1(x, W1, np_nodes)
    spmm_relu = _make_spmm(np_nodes, nch, True)
    spmm_plain = _make_spmm(np_nodes, nch, False)
    h = spmm_relu(sup.reshape(NC * np_nodes, LANES), srcs, dsts, adjs, b1p)
    agg2 = spmm_plain(h.reshape(NC * np_nodes, LANES), srcs, dsts, adjs, b1p)
    return _matmul2(agg2, W3, b3, n)


# speedup vs baseline: 21.0464x; 1.3582x over previous
import functools

import jax
import jax.numpy as jnp
from jax import lax
from jax.experimental import pallas as pl
from jax.experimental.pallas import tpu as pltpu
from jax.experimental.pallas import tpu_sc as plsc

LANES = 16
NT = 16
NC = 2
CH = 512
NSUB = CH // 128


def _matmul1(x, w1, np_nodes):
    m, k = x.shape
    h = w1.shape[1]
    blk = 2048

    def body(x_ref, w_ref, o_ref):
        s = jnp.dot(x_ref[...], w_ref[...], preferred_element_type=jnp.float32)
        o_ref[0] = s[:, :LANES]
        o_ref[1] = s[:, LANES:]

    return pl.pallas_call(
        body,
        grid=(np_nodes // blk,),
        in_specs=[
            pl.BlockSpec((blk, k), lambda i: (i, 0)),
            pl.BlockSpec((k, h), lambda i: (0, 0)),
        ],
        out_specs=pl.BlockSpec((2, blk, LANES), lambda i: (0, i, 0)),
        out_shape=jax.ShapeDtypeStruct((2, np_nodes, LANES), jnp.float32),
    )(x, w1)


def _matmul2(agg, w3, b3, m):
    c = w3.shape[1]
    blk = 2000

    def body(a_ref, w_ref, b_ref, o_ref):
        s = jnp.concatenate([a_ref[0], a_ref[1]], axis=1)
        o_ref[...] = jnp.dot(s, w_ref[...], preferred_element_type=jnp.float32) + b_ref[...]

    return pl.pallas_call(
        body,
        grid=(m // blk,),
        in_specs=[
            pl.BlockSpec((2, blk, LANES), lambda i: (0, i, 0)),
            pl.BlockSpec((2 * LANES, c), lambda i: (0, 0)),
            pl.BlockSpec((1, c), lambda i: (0, 0)),
        ],
        out_specs=pl.BlockSpec((blk, c), lambda i: (i, 0)),
        out_shape=jax.ShapeDtypeStruct((m, c), jnp.float32),
    )(agg, w3, b3.reshape(1, c))


def _make_spmm(np_nodes, nch, fuse_bias_relu):
    assert nch % 2 == 0
    rows_per_tile = np_nodes // NT
    rows_chunk = rows_per_tile // 16
    nrc = 16
    mesh = plsc.VectorSubcoreMesh(core_axis_name="c", subcore_axis_name="s")

    def body(sup_hbm, srcs_hbm, dsts_hbm, adjs_hbm, bias_hbm, out_hbm,
             src_v, dst_v, adj_v, rows_v, post_v, bias_v, acc_sh,
             sem_g, sem_s0, sem_s1, sem_i):
        cid = lax.axis_index("c")
        tid = lax.axis_index("s")
        sem_s = (sem_s0, sem_s1)

        def fire_gathers(b, ci):
            for j in range(NSUB):
                pltpu.async_copy(
                    sup_hbm.at[src_v.at[b, pl.ds(j * 128, 128)]],
                    rows_v.at[b, pl.ds(j * 128, 128)], sem_g)

        def wait_gather(b, j):
            pltpu.make_async_copy(
                sup_hbm.at[src_v.at[b, pl.ds(j * 128, 128)]],
                rows_v.at[b, pl.ds(j * 128, 128)], sem_g).wait()

        def wait_scatters(b):
            for j in range(NSUB):
                pltpu.make_async_copy(rows_v.at[b, pl.ds(j * 128, 128)],
                                      acc_sh.at[dst_v.at[b, j]], sem_s[b]).wait()

        def scale_subchunk(b, j):
            def gbody(g, carry):
                base = j * 128 + g * LANES
                adjv = adj_v[b, pl.ds(base, LANES)]
                for e in range(LANES):
                    rows_v[b, base + e] = rows_v[b, base + e] * adjv[e]
                return carry
            lax.fori_loop(0, 128 // LANES, gbody, None, unroll=2)

        def zbody(i, carry):
            post_v[i] = jnp.zeros((LANES,), jnp.float32)
            return carry
        lax.fori_loop(0, rows_chunk, zbody, None)
        for r in range(nrc):
            pltpu.sync_copy(
                post_v,
                acc_sh.at[pl.ds(tid * rows_per_tile + r * rows_chunk, rows_chunk)])
        plsc.subcore_barrier()

        pltpu.sync_copy(srcs_hbm.at[cid, tid, 0], src_v.at[0])
        pltpu.sync_copy(dsts_hbm.at[tid, 0], dst_v.at[0])
        pltpu.sync_copy(adjs_hbm.at[tid, 0], adj_v.at[0])
        fire_gathers(0, 0)

        def ubody(u, carry):
            for b in (0, 1):
                ob = 1 - b
                ci = u * 2 + b
                if b == 0:
                    @pl.when(u > 0)
                    def _():
                        wait_scatters(ob)
                else:
                    wait_scatters(ob)
                pltpu.async_copy(srcs_hbm.at[cid, tid, ci + 1], src_v.at[ob], sem_i)
                pltpu.async_copy(dsts_hbm.at[tid, ci + 1], dst_v.at[ob], sem_i)
                pltpu.async_copy(adjs_hbm.at[tid, ci + 1], adj_v.at[ob], sem_i)
                for j in range(NSUB):
                    wait_gather(b, j)
                    scale_subchunk(b, j)
                    pltpu.async_copy(rows_v.at[b, pl.ds(j * 128, 128)],
                                     acc_sh.at[dst_v.at[b, j]], sem_s[b], add=True)
                pltpu.make_async_copy(srcs_hbm.at[cid, tid, ci + 1], src_v.at[ob], sem_i).wait()
                pltpu.make_async_copy(dsts_hbm.at[tid, ci + 1], dst_v.at[ob], sem_i).wait()
                pltpu.make_async_copy(adjs_hbm.at[tid, ci + 1], adj_v.at[ob], sem_i).wait()
                fire_gathers(ob, ci + 1)
            return carry
        lax.fori_loop(0, nch // 2, ubody, None)
        for j in range(NSUB):
            wait_gather(0, j)
        wait_scatters(1)
        plsc.subcore_barrier()

        if fuse_bias_relu:
            pltpu.sync_copy(bias_hbm, bias_v)
        for r in range(nrc):
            base = tid * rows_per_tile + r * rows_chunk
            if fuse_bias_relu:
                pltpu.sync_copy(acc_sh.at[pl.ds(base, rows_chunk)], post_v)
                bh = bias_v[cid]

                def pbody(e, carry):
                    post_v[e] = jnp.maximum(post_v[e] + bh, 0.0)
                    return carry
                lax.fori_loop(0, rows_chunk, pbody, None, unroll=8)
                pltpu.sync_copy(post_v, out_hbm.at[cid, pl.ds(base, rows_chunk)])
            else:
                pltpu.sync_copy(acc_sh.at[pl.ds(base, rows_chunk)],
                                out_hbm.at[cid, pl.ds(base, rows_chunk)])

    return pl.kernel(
        body,
        out_type=jax.ShapeDtypeStruct((NC, np_nodes, LANES), jnp.float32),
        mesh=mesh,
        compiler_params=pltpu.CompilerParams(use_tc_tiling_on_sc=False),
        scratch_types=[
            pltpu.VMEM((2, CH), jnp.int32),
            pltpu.VMEM((2, NSUB, 128), jnp.int32),
            pltpu.VMEM((2, CH), jnp.float32),
            pltpu.VMEM((2, CH, LANES), jnp.float32),
            pltpu.VMEM((rows_chunk, LANES), jnp.float32),
            pltpu.VMEM((NC, LANES), jnp.float32),
            pltpu.VMEM_SHARED((np_nodes, LANES), jnp.float32),
            pltpu.SemaphoreType.DMA,
            pltpu.SemaphoreType.DMA,
            pltpu.SemaphoreType.DMA,
            pltpu.SemaphoreType.DMA,
        ],
    )


def kernel(x, edge_index, adj_vals, W1, b1, W3, b3):
    n = x.shape[0]
    e = edge_index.shape[1]
    np_nodes = -(-n // (NT * 32)) * (NT * 32)

    nch = -(-e // (NT * CH))
    nch += nch % 2
    pad = NT * CH * nch - e
    dst = edge_index[0]
    src = edge_index[1]
    if pad:
        src = jnp.pad(src, (0, pad))
        dst = jnp.pad(dst, (0, pad))
        av = jnp.pad(adj_vals, (0, pad))
    else:
        av = adj_vals
    srcs = jnp.pad(jnp.stack([src, src + np_nodes]).reshape(NC, NT, nch, CH),
                   ((0, 0), (0, 0), (0, 1), (0, 0)))
    dsts = jnp.pad(dst.reshape(NT, nch, NSUB, 128), ((0, 0), (0, 1), (0, 0), (0, 0)))
    adjs = jnp.pad(av.reshape(NT, nch, CH), ((0, 0), (0, 1), (0, 0)))
    b1p = b1.reshape(NC, LANES)

    sup = _matmul1(x, W1, np_nodes)
    spmm_relu = _make_spmm(np_nodes, nch, True)
    spmm_plain = _make_spmm(np_nodes, nch, False)
    h = spmm_relu(sup.reshape(NC * np_nodes, LANES), srcs, dsts, adjs, b1p)
    agg2 = spmm_plain(h.reshape(NC * np_nodes, LANES), srcs, dsts, adjs, b1p)
    return _matmul2(agg2, W3, b3, n)
